# baseline (speedup 1.0000x reference)
import functools

import jax
import jax.numpy as jnp
from jax import lax
from jax.experimental import pallas as pl
from jax.experimental.pallas import tpu as pltpu

_BITS_07 = 0x3F333333


def _ohem_body(x_ref, t_ref, seg_ref, segt_ref, m19_ref, out_ref,
               loss_s, pred_s, acc_s, *, nb, kth):
    i = pl.program_id(0)

    @pl.when(i == 0)
    def _init():
        acc_s[...] = jnp.zeros_like(acc_s)

    @pl.when(i < nb)
    def _dense():
        x = x_ref[...]
        bs = x.shape[0]
        t = t_ref[...]
        seg = seg_ref[...]
        segt = segt_ref[...]
        m19 = m19_ref[...]

        e = jnp.exp(x)
        s = lax.dot_general(e, seg, (((1,), (0,)), ((), ())),
                            precision=lax.Precision.HIGHEST,
                            preferred_element_type=jnp.float32)
        t_exp = lax.dot_general(t, segt, (((1,), (0,)), ((), ())),
                                precision=lax.Precision.DEFAULT,
                                preferred_element_type=jnp.float32)
        xm = jnp.where(t_exp == m19, x, 0.0)
        tx = lax.dot_general(xm, seg, (((1,), (0,)), ((), ())),
                             precision=lax.Precision.HIGHEST,
                             preferred_element_type=jnp.float32)
        loss = jnp.log(s) - tx
        pred = jnp.exp(tx) / s
        loss_s[pl.ds(i * bs, bs), :] = loss
        pred_s[pl.ds(i * bs, bs), :] = pred
        keep = pred < 0.7
        acc_s[0:1, :] += jnp.sum(jnp.where(keep, loss, 0.0), axis=0,
                                 keepdims=True)
        acc_s[1:2, :] += jnp.sum(keep.astype(jnp.float32), axis=0,
                                 keepdims=True)

    @pl.when(i == nb)
    def _select():
        c07 = jnp.sum(acc_s[1:2, :])

        def _fast(_):
            return jnp.sum(acc_s[0:1, :]) / c07

        def _search(_):
            bits = lax.bitcast_convert_type(pred_s[...], jnp.int32)

            def bs_body(_, carry):
                lo, hi = carry
                mid = lax.div(lo + hi, 2)
                cnt = jnp.sum((bits <= mid).astype(jnp.int32))
                geq = cnt >= kth + 1
                return (jnp.where(geq, lo, mid + 1), jnp.where(geq, mid, hi))

            lo, _ = lax.fori_loop(0, 31, bs_body,
                                  (jnp.int32(0), jnp.int32(1 << 30)))
            thr = jnp.maximum(lo, _BITS_07)
            keep = bits < thr
            num = jnp.sum(jnp.where(keep, loss_s[...], 0.0))
            den = jnp.sum(keep.astype(jnp.float32))
            return num / den

        result = lax.cond(c07 > jnp.float32(kth), _fast, _search, None)
        out_ref[...] = result[None, None]


def kernel(score, target):
    n, c = score.shape
    lanes = 128
    srows = n // lanes
    width = c * lanes
    bs = 256
    nb = srows // bs
    kth = min(int(0.7 * n), n - 1)

    x_flat = score.reshape(srows, width)
    t2 = target.reshape(srows, lanes).astype(jnp.float32)
    r_idx = jnp.arange(width, dtype=jnp.int32)
    seg = (r_idx[:, None] // c == jnp.arange(lanes)[None, :]).astype(jnp.float32)
    segt = seg.T
    m19 = (r_idx % c).astype(jnp.float32).reshape(1, width)

    out = pl.pallas_call(
        functools.partial(_ohem_body, nb=nb, kth=kth),
        grid=(nb + 1,),
        in_specs=[
            pl.BlockSpec((bs, width), lambda i: (jnp.minimum(i, nb - 1), 0)),
            pl.BlockSpec((bs, lanes), lambda i: (jnp.minimum(i, nb - 1), 0)),
            pl.BlockSpec((width, lanes), lambda i: (0, 0)),
            pl.BlockSpec((lanes, width), lambda i: (0, 0)),
            pl.BlockSpec((1, width), lambda i: (0, 0)),
        ],
        out_specs=pl.BlockSpec((1, 1), lambda i: (0, 0)),
        out_shape=jax.ShapeDtypeStruct((1, 1), jnp.float32),
        scratch_shapes=[
            pltpu.VMEM((srows, lanes), jnp.float32),
            pltpu.VMEM((srows, lanes), jnp.float32),
            pltpu.VMEM((2, lanes), jnp.float32),
        ],
    )(x_flat, t2, seg, segt, m19)
    return out[0, 0]


# speedup vs baseline: 1.2705x; 1.2705x over previous
import functools

import jax
import jax.numpy as jnp
from jax import lax
from jax.experimental import pallas as pl
from jax.experimental.pallas import tpu as pltpu

_BITS_07 = 0x3F333333


def _ohem_body(x_ref, t_ref, seg_ref, segt_ref, m19_ref, out_ref,
               loss_s, pred_s, acc_s, *, nb, kth):
    i = pl.program_id(0)

    @pl.when(i == 0)
    def _init():
        acc_s[...] = jnp.zeros_like(acc_s)

    @pl.when(i < nb)
    def _dense():
        x = x_ref[...]
        bs = x.shape[0]
        t = t_ref[...]
        seg = seg_ref[...]
        segt = segt_ref[...]
        m19 = m19_ref[...]

        def seg_dot(a):
            a_hi = a.astype(jnp.bfloat16)
            a_lo = (a - a_hi.astype(jnp.float32)).astype(jnp.bfloat16)
            dot = lambda m: lax.dot_general(
                m, seg, (((1,), (0,)), ((), ())),
                precision=lax.Precision.DEFAULT,
                preferred_element_type=jnp.float32)
            return dot(a_hi) + dot(a_lo)

        e = jnp.exp(x)
        s = seg_dot(e)
        t_exp = lax.dot_general(t, segt, (((1,), (0,)), ((), ())),
                                precision=lax.Precision.DEFAULT,
                                preferred_element_type=jnp.float32)
        xm = jnp.where(t_exp == m19, x, 0.0)
        tx = seg_dot(xm)
        loss = jnp.log(s) - tx
        pred = jnp.exp(tx) / s
        loss_s[pl.ds(i * bs, bs), :] = loss
        pred_s[pl.ds(i * bs, bs), :] = pred
        keep = pred < 0.7
        acc_s[0:1, :] += jnp.sum(jnp.where(keep, loss, 0.0), axis=0,
                                 keepdims=True)
        acc_s[1:2, :] += jnp.sum(keep.astype(jnp.float32), axis=0,
                                 keepdims=True)

    @pl.when(i == nb)
    def _select():
        c07 = jnp.sum(acc_s[1:2, :])

        def _fast(_):
            return jnp.sum(acc_s[0:1, :]) / c07

        def _search(_):
            bits = lax.bitcast_convert_type(pred_s[...], jnp.int32)

            def bs_body(_, carry):
                lo, hi = carry
                mid = lax.div(lo + hi, 2)
                cnt = jnp.sum((bits <= mid).astype(jnp.int32))
                geq = cnt >= kth + 1
                return (jnp.where(geq, lo, mid + 1), jnp.where(geq, mid, hi))

            lo, _ = lax.fori_loop(0, 31, bs_body,
                                  (jnp.int32(0), jnp.int32(1 << 30)))
            thr = jnp.maximum(lo, _BITS_07)
            keep = bits < thr
            num = jnp.sum(jnp.where(keep, loss_s[...], 0.0))
            den = jnp.sum(keep.astype(jnp.float32))
            return num / den

        result = lax.cond(c07 > jnp.float32(kth), _fast, _search, None)
        out_ref[...] = result[None, None]


def kernel(score, target):
    n, c = score.shape
    lanes = 128
    srows = n // lanes
    width = c * lanes
    bs = 256
    nb = srows // bs
    kth = min(int(0.7 * n), n - 1)

    x_flat = score.reshape(srows, width)
    t2 = target.reshape(srows, lanes).astype(jnp.bfloat16)
    r_idx = jnp.arange(width, dtype=jnp.int32)
    seg = (r_idx[:, None] // c == jnp.arange(lanes)[None, :]).astype(jnp.bfloat16)
    segt = seg.T
    m19 = (r_idx % c).astype(jnp.float32).reshape(1, width)

    out = pl.pallas_call(
        functools.partial(_ohem_body, nb=nb, kth=kth),
        grid=(nb + 1,),
        in_specs=[
            pl.BlockSpec((bs, width), lambda i: (jnp.minimum(i, nb - 1), 0)),
            pl.BlockSpec((bs, lanes), lambda i: (jnp.minimum(i, nb - 1), 0)),
            pl.BlockSpec((width, lanes), lambda i: (0, 0)),
            pl.BlockSpec((lanes, width), lambda i: (0, 0)),
            pl.BlockSpec((1, width), lambda i: (0, 0)),
        ],
        out_specs=pl.BlockSpec((1, 1), lambda i: (0, 0)),
        out_shape=jax.ShapeDtypeStruct((1, 1), jnp.float32),
        scratch_shapes=[
            pltpu.VMEM((srows, lanes), jnp.float32),
            pltpu.VMEM((srows, lanes), jnp.float32),
            pltpu.VMEM((2, lanes), jnp.float32),
        ],
    )(x_flat, t2, seg, segt, m19)
    return out[0, 0]


# speedup vs baseline: 1.2767x; 1.0049x over previous
import functools

import numpy as np

import jax
import jax.numpy as jnp
from jax import lax
from jax.experimental import pallas as pl
from jax.experimental.pallas import tpu as pltpu

_BITS_07 = 0x3F333333


def _ohem_body(x_ref, t_ref, seg_ref, segt_ref, m19_ref, out_ref,
               loss_s, pred_s, acc_s, *, nb, kth):
    i = pl.program_id(0)

    @pl.when(i == 0)
    def _init():
        acc_s[...] = jnp.zeros_like(acc_s)

    @pl.when(i < nb)
    def _dense():
        x = x_ref[...]
        bs = x.shape[0]
        t = t_ref[...].astype(jnp.bfloat16)
        seg = seg_ref[...]
        segt = segt_ref[...]
        m19 = m19_ref[...]

        def seg_dot(a):
            a_hi = a.astype(jnp.bfloat16)
            a_lo = (a - a_hi.astype(jnp.float32)).astype(jnp.bfloat16)
            dot = lambda m: lax.dot_general(
                m, seg, (((1,), (0,)), ((), ())),
                precision=lax.Precision.DEFAULT,
                preferred_element_type=jnp.float32)
            return dot(a_hi) + dot(a_lo)

        e = jnp.exp(x)
        s = seg_dot(e)
        t_exp = lax.dot_general(t, segt, (((1,), (0,)), ((), ())),
                                precision=lax.Precision.DEFAULT,
                                preferred_element_type=jnp.float32)
        xm = jnp.where(t_exp == m19, x, 0.0)
        tx = seg_dot(xm)
        loss = jnp.log(s) - tx
        pred = jnp.exp(tx) / s
        loss_s[pl.ds(i * bs, bs), :] = loss
        pred_s[pl.ds(i * bs, bs), :] = pred
        keep = pred < 0.7
        acc_s[0:1, :] += jnp.sum(jnp.where(keep, loss, 0.0), axis=0,
                                 keepdims=True)
        acc_s[1:2, :] += jnp.sum(keep.astype(jnp.float32), axis=0,
                                 keepdims=True)

    @pl.when(i == nb)
    def _select():
        c07 = jnp.sum(acc_s[1:2, :])

        def _fast(_):
            return jnp.sum(acc_s[0:1, :]) / c07

        def _search(_):
            bits = lax.bitcast_convert_type(pred_s[...], jnp.int32)

            def bs_body(_, carry):
                lo, hi = carry
                mid = lax.div(lo + hi, 2)
                cnt = jnp.sum((bits <= mid).astype(jnp.int32))
                geq = cnt >= kth + 1
                return (jnp.where(geq, lo, mid + 1), jnp.where(geq, mid, hi))

            lo, _ = lax.fori_loop(0, 31, bs_body,
                                  (jnp.int32(0), jnp.int32(1 << 30)))
            thr = jnp.maximum(lo, _BITS_07)
            keep = bits < thr
            num = jnp.sum(jnp.where(keep, loss_s[...], 0.0))
            den = jnp.sum(keep.astype(jnp.float32))
            return num / den

        result = lax.cond(c07 > jnp.float32(kth), _fast, _search, None)
        out_ref[...] = result[None, None]


def kernel(score, target):
    n, c = score.shape
    lanes = 128
    srows = n // lanes
    width = c * lanes
    bs = 256
    nb = srows // bs
    kth = min(int(0.7 * n), n - 1)

    x_flat = score.reshape(srows, width)
    t2 = target.reshape(srows, lanes)
    r_idx = np.arange(width, dtype=np.int32)
    seg_np = (r_idx[:, None] // c == np.arange(lanes)[None, :]).astype(np.float32)
    seg = jnp.asarray(seg_np, dtype=jnp.bfloat16)
    segt = jnp.asarray(seg_np.T, dtype=jnp.bfloat16)
    m19 = jnp.asarray((r_idx % c).astype(np.float32).reshape(1, width))

    out = pl.pallas_call(
        functools.partial(_ohem_body, nb=nb, kth=kth),
        grid=(nb + 1,),
        in_specs=[
            pl.BlockSpec((bs, width), lambda i: (jnp.minimum(i, nb - 1), 0)),
            pl.BlockSpec((bs, lanes), lambda i: (jnp.minimum(i, nb - 1), 0)),
            pl.BlockSpec((width, lanes), lambda i: (0, 0)),
            pl.BlockSpec((lanes, width), lambda i: (0, 0)),
            pl.BlockSpec((1, width), lambda i: (0, 0)),
        ],
        out_specs=pl.BlockSpec((1, 1), lambda i: (0, 0)),
        out_shape=jax.ShapeDtypeStruct((1, 1), jnp.float32),
        scratch_shapes=[
            pltpu.VMEM((srows, lanes), jnp.float32),
            pltpu.VMEM((srows, lanes), jnp.float32),
            pltpu.VMEM((2, lanes), jnp.float32),
        ],
    )(x_flat, t2, seg, segt, m19)
    return out[0, 0]
